# initial kernel scaffold (unmeasured)
import jax
import jax.numpy as jnp
from jax import lax
from jax.experimental import pallas as pl
from jax.experimental.pallas import tpu as pltpu


def kernel(
    x,
):
    def body(*refs):
        pass

    out_shape = jax.ShapeDtypeStruct(..., jnp.float32)
    return pl.pallas_call(body, out_shape=out_shape)(...)


# baseline (device time: 17811 ns/iter reference)
import jax
import jax.numpy as jnp
from jax import lax
from jax.experimental import pallas as pl
from jax.experimental.pallas import tpu as pltpu

N_Y = 2


def kernel(x):
    m, n = x.shape

    def body(x_ref, out_ref, send_sem, recv_sem):
        my_x = lax.axis_index("x")
        my_y = lax.axis_index("y")
        nbr = (my_x, 1 - my_y)

        barrier = pltpu.get_barrier_semaphore()
        pl.semaphore_signal(
            barrier, inc=1, device_id=nbr, device_id_type=pl.DeviceIdType.MESH
        )
        pl.semaphore_wait(barrier, 1)

        out_ref[pl.ds(my_y * m, m), :] = x_ref[...].astype(jnp.bfloat16)

        rdma = pltpu.make_async_remote_copy(
            src_ref=out_ref.at[pl.ds(my_y * m, m), :],
            dst_ref=out_ref.at[pl.ds(my_y * m, m), :],
            send_sem=send_sem,
            recv_sem=recv_sem,
            device_id=nbr,
            device_id_type=pl.DeviceIdType.MESH,
        )
        rdma.start()
        rdma.wait()

    return pl.pallas_call(
        body,
        out_shape=jax.ShapeDtypeStruct((N_Y * m, n), jnp.bfloat16),
        in_specs=[pl.BlockSpec(memory_space=pltpu.VMEM)],
        out_specs=pl.BlockSpec(memory_space=pltpu.VMEM),
        scratch_shapes=[
            pltpu.SemaphoreType.DMA,
            pltpu.SemaphoreType.DMA,
        ],
        compiler_params=pltpu.CompilerParams(collective_id=0),
    )(x)


# device time: 16116 ns/iter; 1.1052x vs baseline; 1.1052x over previous
import jax
import jax.numpy as jnp
from jax import lax
from jax.experimental import pallas as pl
from jax.experimental.pallas import tpu as pltpu

N_Y = 2
K = 4


def kernel(x):
    m, n = x.shape
    half = m // 2
    sub = half // K

    def body(x_ref, out_ref, ysend_sems, yrecv_sems, xsend_sems, xrecv_sems):
        my_x = lax.axis_index("x")
        my_y = lax.axis_index("y")
        nbr_y = (my_x, 1 - my_y)
        nbr_x = (1 - my_x, my_y)

        barrier = pltpu.get_barrier_semaphore()
        for nbr in (nbr_y, nbr_x):
            pl.semaphore_signal(
                barrier, inc=1, device_id=nbr,
                device_id_type=pl.DeviceIdType.MESH,
            )
        pl.semaphore_wait(barrier, 2)

        gy = my_y * m
        send_base = gy + my_x * half
        keep_base = gy + (1 - my_x) * half
        yrecv_base = (1 - my_y) * m + my_x * half
        xrecv_base = (1 - my_y) * m + (1 - my_x) * half

        y_sends = []
        for k in range(K):
            rows = pl.ds(send_base + k * sub, sub)
            out_ref[rows, :] = x_ref[
                pl.ds(my_x * half + k * sub, sub), :
            ].astype(jnp.bfloat16)
            r = pltpu.make_async_remote_copy(
                src_ref=out_ref.at[rows, :],
                dst_ref=out_ref.at[rows, :],
                send_sem=ysend_sems.at[k],
                recv_sem=yrecv_sems.at[k],
                device_id=nbr_y,
                device_id_type=pl.DeviceIdType.MESH,
            )
            r.start()
            y_sends.append(r)

        out_ref[pl.ds(keep_base, half), :] = x_ref[
            pl.ds((1 - my_x) * half, half), :
        ].astype(jnp.bfloat16)

        x_sends = []
        for k in range(K):
            rows = pl.ds(yrecv_base + k * sub, sub)
            recv = pltpu.make_async_remote_copy(
                src_ref=out_ref.at[rows, :],
                dst_ref=out_ref.at[rows, :],
                send_sem=ysend_sems.at[k],
                recv_sem=yrecv_sems.at[k],
                device_id=nbr_y,
                device_id_type=pl.DeviceIdType.MESH,
            )
            recv.wait_recv()
            r = pltpu.make_async_remote_copy(
                src_ref=out_ref.at[rows, :],
                dst_ref=out_ref.at[rows, :],
                send_sem=xsend_sems.at[k],
                recv_sem=xrecv_sems.at[k],
                device_id=nbr_x,
                device_id_type=pl.DeviceIdType.MESH,
            )
            r.start()
            x_sends.append(r)

        for k in range(K):
            rows = pl.ds(xrecv_base + k * sub, sub)
            recv = pltpu.make_async_remote_copy(
                src_ref=out_ref.at[rows, :],
                dst_ref=out_ref.at[rows, :],
                send_sem=xsend_sems.at[k],
                recv_sem=xrecv_sems.at[k],
                device_id=nbr_x,
                device_id_type=pl.DeviceIdType.MESH,
            )
            recv.wait_recv()

        for r in y_sends:
            r.wait_send()
        for r in x_sends:
            r.wait_send()

    return pl.pallas_call(
        body,
        out_shape=jax.ShapeDtypeStruct((N_Y * m, n), jnp.bfloat16),
        in_specs=[pl.BlockSpec(memory_space=pltpu.VMEM)],
        out_specs=pl.BlockSpec(memory_space=pltpu.VMEM),
        scratch_shapes=[
            pltpu.SemaphoreType.DMA((K,)),
            pltpu.SemaphoreType.DMA((K,)),
            pltpu.SemaphoreType.DMA((K,)),
            pltpu.SemaphoreType.DMA((K,)),
        ],
        compiler_params=pltpu.CompilerParams(collective_id=0),
    )(x)


# device time: 15543 ns/iter; 1.1459x vs baseline; 1.0369x over previous
import jax
import jax.numpy as jnp
from jax import lax
from jax.experimental import pallas as pl
from jax.experimental.pallas import tpu as pltpu

N_Y = 2
K = 8


def kernel(x):
    m, n = x.shape
    half = m // 2
    sub = half // K

    def body(x_ref, out_ref, ysend_sems, yrecv_sems, xsend_sems, xrecv_sems):
        my_x = lax.axis_index("x")
        my_y = lax.axis_index("y")
        nbr_y = (my_x, 1 - my_y)
        nbr_x = (1 - my_x, my_y)

        barrier = pltpu.get_barrier_semaphore()
        for nbr in (nbr_y, nbr_x):
            pl.semaphore_signal(
                barrier, inc=1, device_id=nbr,
                device_id_type=pl.DeviceIdType.MESH,
            )
        pl.semaphore_wait(barrier, 2)

        gy = my_y * m
        send_base = gy + my_x * half
        keep_base = gy + (1 - my_x) * half
        yrecv_base = (1 - my_y) * m + my_x * half
        xrecv_base = (1 - my_y) * m + (1 - my_x) * half

        y_sends = []
        for k in range(K):
            rows = pl.ds(send_base + k * sub, sub)
            out_ref[rows, :] = x_ref[
                pl.ds(my_x * half + k * sub, sub), :
            ].astype(jnp.bfloat16)
            r = pltpu.make_async_remote_copy(
                src_ref=out_ref.at[rows, :],
                dst_ref=out_ref.at[rows, :],
                send_sem=ysend_sems.at[k],
                recv_sem=yrecv_sems.at[k],
                device_id=nbr_y,
                device_id_type=pl.DeviceIdType.MESH,
            )
            r.start()
            y_sends.append(r)

        out_ref[pl.ds(keep_base, half), :] = x_ref[
            pl.ds((1 - my_x) * half, half), :
        ].astype(jnp.bfloat16)

        x_sends = []
        for k in range(K):
            rows = pl.ds(yrecv_base + k * sub, sub)
            recv = pltpu.make_async_remote_copy(
                src_ref=out_ref.at[rows, :],
                dst_ref=out_ref.at[rows, :],
                send_sem=ysend_sems.at[k],
                recv_sem=yrecv_sems.at[k],
                device_id=nbr_y,
                device_id_type=pl.DeviceIdType.MESH,
            )
            recv.wait_recv()
            r = pltpu.make_async_remote_copy(
                src_ref=out_ref.at[rows, :],
                dst_ref=out_ref.at[rows, :],
                send_sem=xsend_sems.at[k],
                recv_sem=xrecv_sems.at[k],
                device_id=nbr_x,
                device_id_type=pl.DeviceIdType.MESH,
            )
            r.start()
            x_sends.append(r)

        for k in range(K):
            rows = pl.ds(xrecv_base + k * sub, sub)
            recv = pltpu.make_async_remote_copy(
                src_ref=out_ref.at[rows, :],
                dst_ref=out_ref.at[rows, :],
                send_sem=xsend_sems.at[k],
                recv_sem=xrecv_sems.at[k],
                device_id=nbr_x,
                device_id_type=pl.DeviceIdType.MESH,
            )
            recv.wait_recv()

        for r in y_sends:
            r.wait_send()
        for r in x_sends:
            r.wait_send()

    return pl.pallas_call(
        body,
        out_shape=jax.ShapeDtypeStruct((N_Y * m, n), jnp.bfloat16),
        in_specs=[pl.BlockSpec(memory_space=pltpu.VMEM)],
        out_specs=pl.BlockSpec(memory_space=pltpu.VMEM),
        scratch_shapes=[
            pltpu.SemaphoreType.DMA((K,)),
            pltpu.SemaphoreType.DMA((K,)),
            pltpu.SemaphoreType.DMA((K,)),
            pltpu.SemaphoreType.DMA((K,)),
        ],
        compiler_params=pltpu.CompilerParams(collective_id=0),
    )(x)
